# baseline (speedup 1.0000x reference)
import functools

import jax
import jax.numpy as jnp
from jax import lax
from jax.experimental import pallas as pl
from jax.experimental.pallas import tpu as pltpu, tpu_sc as plsc

N = 10000
NPAD = 10240
IN_CH = 128
HIDDEN = 256
HEADS = 8
DH = HIDDEN // HEADS
NEG_SLOPE = 0.01
EPS = 1e-5
NB = 64

E = 320000
NPT = 80
NRANGES = NPAD // NPT
C = 96
MCH = 16 * C
EPAD = 323136
INV_SQRT_DH = 1.0 / (DH ** 0.5)

NC = 2
NS = 16

import numpy as _np
_PERM = _np.array([(l % 8) * 32 + 2 * j + (l // 8)
                   for j in range(16) for l in range(16)], dtype=_np.int32)
_IPERM = _np.argsort(_PERM).astype(_np.int32)
_PERM_BF = _np.empty(256, dtype=_np.int32)
for _p in range(8):
    for _t in range(16):
        _PERM_BF[32 * _p + 2 * _t] = _PERM[16 * (2 * _p) + _t]
        _PERM_BF[32 * _p + 2 * _t + 1] = _PERM[16 * (2 * _p + 1) + _t]


def _sload(ref, i):
    return ref[pl.ds(i, 16)][0]


def _gather16(vec, idx):
    return lax.gather(
        vec, idx.reshape(16, 1),
        lax.GatherDimensionNumbers(offset_dims=(), collapsed_slice_dims=(0,),
                                   start_index_map=(0,)),
        (1,), mode=lax.GatherScatterMode.PROMISE_IN_BOUNDS)


def _take16(vec, h):
    return _gather16(vec, jnp.full((16,), h, jnp.int32))


def _unpack_bf16(xi):
    a = lax.bitcast_convert_type(lax.shift_left(xi, 16), jnp.float32)
    b = lax.bitcast_convert_type(jnp.bitwise_and(xi, jnp.int32(-65536)),
                                 jnp.float32)
    return a, b


def _allsum16(vec, perms):
    t = vec
    for p in perms:
        t = t + _gather16(t, p)
    return t


def _mm_body(x_ref, w_ref, b_ref, qo, ko, vo, so):
    r = jnp.dot(x_ref[...], w_ref[...], preferred_element_type=jnp.float32)
    r = r + b_ref[...]
    qo[...] = r[:, 0:256]
    ko[...] = r[:, 256:512].astype(jnp.bfloat16)
    vo[...] = r[:, 512:768].astype(jnp.bfloat16)
    so[...] = r[:, 768:1024]


def _project(x, wall, ball):
    blk = 512
    grid = (NPAD // blk,)
    out = pl.pallas_call(
        _mm_body,
        grid=grid,
        in_specs=[
            pl.BlockSpec((blk, IN_CH), lambda i: (i, 0)),
            pl.BlockSpec((IN_CH, 4 * HIDDEN), lambda i: (0, 0)),
            pl.BlockSpec((1, 4 * HIDDEN), lambda i: (0, 0)),
        ],
        out_specs=[pl.BlockSpec((blk, HIDDEN), lambda i: (i, 0))] * 4,
        out_shape=[jax.ShapeDtypeStruct((NPAD, HIDDEN), jnp.float32),
                   jax.ShapeDtypeStruct((NPAD, HIDDEN), jnp.bfloat16),
                   jax.ShapeDtypeStruct((NPAD, HIDDEN), jnp.bfloat16),
                   jax.ShapeDtypeStruct((NPAD, HIDDEN), jnp.float32)],
    )(x, wall, ball)
    return out


def _wid():
    return lax.axis_index("s") * NC + lax.axis_index("c")


def _sc_edge_body(q_hbm, k_hbm, v_hbm, src2_hbm, dw_hbm, elo_hbm,
                  nmc_hbm, out_hbm, q_loc, num, den, krows0, vrows0, krows1,
                  vrows1, srcv2, dwv, elov, nmcv, semk0, semv0, semk1, semv1):
    wid = _wid()
    pltpu.sync_copy(elo_hbm, elov.at[pl.ds(0, NRANGES)])
    pltpu.sync_copy(nmc_hbm, nmcv.at[pl.ds(0, NRANGES)])
    iota16 = lax.iota(jnp.int32, 16)
    zero16 = jnp.zeros((16,), jnp.float32)
    perm8 = iota16 ^ 8
    kbufs = (krows0, krows1)
    vbufs = (vrows0, vrows1)
    ksems = (semk0, semk1)
    vsems = (semv0, semv1)

    def range_body(rr, _):
        r = wid * 4 + rr
        nlo = r * NPT

        def zero_body(n, _z):
            for j in range(16):
                num[n, pl.ds(j * 16, 16)] = zero16
            den[n, :] = zero16
            return 0
        lax.fori_loop(0, NPT, zero_body, 0)

        pltpu.sync_copy(q_hbm.at[pl.ds(pl.multiple_of(nlo, 8), NPT)], q_loc)

        elo = _sload(elov, r)
        nmc = _sload(nmcv, r)

        def make_edge_body(c, kb, vb):
            def pair_body(ei, _z):
                dww = dwv[pl.ds(2 * (c * C) + 4 * ei, 16)]
                for half in range(2):
                    e = 2 * ei + half
                    dstl_raw = dww[2 * half] - nlo
                    w_e = dww[2 * half + 1].astype(jnp.float32)
                    inr = jnp.logical_and(dstl_raw >= 0, dstl_raw < NPT)
                    dstl = jnp.clip(dstl_raw, 0, NPT - 1)
                    wf = jnp.where(inr, w_e, 0.0)

                    prods = [None] * 16
                    for p in range(8):
                        ka, kbb = _unpack_bf16(
                            kb[e, pl.ds(p * 16, 16)])
                        prods[2 * p] = ka * q_loc[dstl,
                                                  pl.ds((2 * p) * 16, 16)]
                        prods[2 * p + 1] = kbb * q_loc[
                            dstl, pl.ds((2 * p + 1) * 16, 16)]
                    while len(prods) > 1:
                        prods = [prods[i] + prods[i + 1]
                                 for i in range(0, len(prods), 2)]
                    part = prods[0]
                    dup = part + _gather16(part, perm8)
                    evec = jnp.exp(dup * INV_SQRT_DH) * wf
                    den[dstl, :] = den[dstl, :] + evec
                    for p in range(8):
                        va, vbb = _unpack_bf16(
                            vb[e, pl.ds(p * 16, 16)])
                        j0 = 2 * p
                        num[dstl, pl.ds(j0 * 16, 16)] = (
                            num[dstl, pl.ds(j0 * 16, 16)] + evec * va)
                        num[dstl, pl.ds((j0 + 1) * 16, 16)] = (
                            num[dstl, pl.ds((j0 + 1) * 16, 16)] + evec * vbb)
                return 0
            return pair_body

        def meta_body(m, _z):
            moff = pl.multiple_of(elo + m * MCH, 768)
            mrow = pl.multiple_of(moff // C, 8)
            pltpu.sync_copy(dw_hbm.at[pl.ds(pl.multiple_of(2 * moff, 8),
                                            2 * MCH)],
                            dwv.at[pl.ds(0, 2 * MCH)])
            pltpu.sync_copy(src2_hbm.at[pl.ds(mrow, 16)], srcv2)
            pend = [None, None]
            for c in range(16):
                b = c % 2
                if c == 0:
                    pend[0] = (
                        pltpu.async_copy(k_hbm.at[srcv2.at[0]], kbufs[0],
                                         ksems[0]),
                        pltpu.async_copy(v_hbm.at[srcv2.at[0]], vbufs[0],
                                         vsems[0]),
                    )
                if c < 15:
                    nb = (c + 1) % 2
                    pend[nb] = (
                        pltpu.async_copy(k_hbm.at[srcv2.at[c + 1]], kbufs[nb],
                                         ksems[nb]),
                        pltpu.async_copy(v_hbm.at[srcv2.at[c + 1]], vbufs[nb],
                                         vsems[nb]),
                    )
                pend[b][0].wait()
                pend[b][1].wait()
                lax.fori_loop(0, C // 2, make_edge_body(c, kbufs[b], vbufs[b]), 0)
            return 0

        lax.fori_loop(0, nmc, meta_body, 0)

        def div_body(n, _z):
            inv = 1.0 / (den[n, :] + 1e-16)
            for j in range(16):
                num[n, pl.ds(j * 16, 16)] = num[n, pl.ds(j * 16, 16)] * inv
            return 0
        lax.fori_loop(0, NPT, div_body, 0)

        pltpu.sync_copy(num, out_hbm.at[pl.ds(pl.multiple_of(nlo, 8), NPT)])
        return 0

    lax.fori_loop(0, 4, range_body, 0)


def _sc_edge(q, k, v, src2, dw, elo, nmc, interpret=False):
    mesh = plsc.VectorSubcoreMesh(core_axis_name="c", subcore_axis_name="s",
                                  num_cores=NC, num_subcores=NS)
    f = pl.kernel(
        _sc_edge_body,
        out_type=jax.ShapeDtypeStruct((NPAD, HIDDEN), jnp.float32),
        mesh=mesh,
        scratch_types=[
            pltpu.VMEM((NPT, HIDDEN), jnp.float32),
            pltpu.VMEM((NPT, HIDDEN), jnp.float32),
            pltpu.VMEM((NPT, 16), jnp.float32),
            pltpu.VMEM((C, 128), jnp.int32),
            pltpu.VMEM((C, 128), jnp.int32),
            pltpu.VMEM((C, 128), jnp.int32),
            pltpu.VMEM((C, 128), jnp.int32),
            pltpu.VMEM((16, C), jnp.int32),
            pltpu.VMEM((2 * MCH + 16,), jnp.int32),
            pltpu.VMEM((NRANGES + 16,), jnp.int32),
            pltpu.VMEM((NRANGES + 16,), jnp.int32),
            pltpu.SemaphoreType.DMA,
            pltpu.SemaphoreType.DMA,
            pltpu.SemaphoreType.DMA,
            pltpu.SemaphoreType.DMA,
        ],
        interpret=interpret,
    )
    return f(q, k, v, src2, dw, elo, nmc)


FBLK = 1024
NFB = NPAD // FBLK


def _final_body(batch_ref, msg_ref, skip_ref, lnw_ref, lnb_ref, out_ref,
                s1, s2, aux):
    p = pl.program_id(0)
    i = pl.program_id(1)

    @pl.when(jnp.logical_and(p == 0, i == 0))
    def _init():
        s1[...] = jnp.zeros_like(s1)
        s2[...] = jnp.zeros_like(s2)
        aux[...] = jnp.zeros_like(aux)

    out = msg_ref[...] + skip_ref[...]
    oh = (batch_ref[...] == lax.broadcasted_iota(jnp.int32, (FBLK, NB), 1))
    oh = oh.astype(jnp.float32)

    @pl.when(p == 0)
    def _pass0():
        s1[...] = s1[...] + lax.dot_general(
            oh, out, (((0,), (0,)), ((), ())),
            preferred_element_type=jnp.float32)
        s2[...] = s2[...] + lax.dot_general(
            oh, out * out, (((0,), (0,)), ((), ())),
            preferred_element_type=jnp.float32)
        aux[:, 0:1] = aux[:, 0:1] + jnp.sum(oh, axis=0)[:, None]

    @pl.when(jnp.logical_and(p == 0, i == NFB - 1))
    def _stats():
        cnt = aux[:, 0:1]
        normc = jnp.clip(cnt, 1.0, None) * float(HIDDEN)
        mean = jnp.sum(s1[...], axis=1)[:, None] / normc
        ms = jnp.sum(s2[...], axis=1)[:, None] / normc
        var = ms - mean * mean
        inv = lax.rsqrt(var + EPS)
        aux[:, 1:2] = mean
        aux[:, 2:3] = inv
        s2[...] = jnp.zeros_like(s2)

    @pl.when(p == 1)
    def _pass1():
        mean_row = jnp.dot(oh, aux[:, 1:2],
                           preferred_element_type=jnp.float32)
        inv_row = jnp.dot(oh, aux[:, 2:3],
                          preferred_element_type=jnp.float32)
        y = (out - mean_row) * inv_row
        y = y * lnw_ref[...] + lnb_ref[...]
        y = jnp.where(y >= 0, y, NEG_SLOPE * y)
        s2[...] = s2[...] + lax.dot_general(
            oh, y, (((0,), (0,)), ((), ())),
            preferred_element_type=jnp.float32)

    @pl.when(jnp.logical_and(p == 1, i == NFB - 1))
    def _emit():
        out_ref[...] = s2[...] / jnp.clip(aux[:, 0:1], 1.0, None)


def _final(msg, skip, batch_pad, lnw, lnb):
    return pl.pallas_call(
        _final_body,
        grid=(2, NFB),
        in_specs=[
            pl.BlockSpec((FBLK, 1), lambda p, i: (i, 0)),
            pl.BlockSpec((FBLK, HIDDEN), lambda p, i: (i, 0)),
            pl.BlockSpec((FBLK, HIDDEN), lambda p, i: (i, 0)),
            pl.BlockSpec((1, HIDDEN), lambda p, i: (0, 0)),
            pl.BlockSpec((1, HIDDEN), lambda p, i: (0, 0)),
        ],
        out_specs=pl.BlockSpec((NB, HIDDEN), lambda p, i: (0, 0)),
        out_shape=jax.ShapeDtypeStruct((NB, HIDDEN), jnp.float32),
        scratch_shapes=[
            pltpu.VMEM((NB, HIDDEN), jnp.float32),
            pltpu.VMEM((NB, HIDDEN), jnp.float32),
            pltpu.VMEM((NB, 128), jnp.float32),
        ],
    )(batch_pad, msg, skip, lnw, lnb)


def kernel(feat1, feat2, edge_index1, edge_index2, batch, Wq, bq, Wk, bk,
           Wv, bv, Wskip, bskip, ln_w, ln_b):
    ei = jnp.concatenate([edge_index1, edge_index2], axis=1)
    code = ei[1] * jnp.int32(N) + ei[0]
    scode = jnp.sort(code)
    keep = jnp.concatenate(
        [jnp.ones((1,), bool), scode[1:] != scode[:-1]])
    src = scode % jnp.int32(N)
    dst = scode // jnp.int32(N)
    w = keep.astype(jnp.float32)
    padn = EPAD - E
    src_p = jnp.concatenate([src, jnp.zeros((padn,), jnp.int32)])
    dst_p = jnp.concatenate([dst, jnp.full((padn,), N, jnp.int32)])
    w_i = jnp.concatenate([keep.astype(jnp.int32),
                           jnp.zeros((padn,), jnp.int32)])
    src2 = src_p.reshape(EPAD // C, C)
    dw = jnp.stack([dst_p, w_i], axis=1).reshape(2 * EPAD)
    bounds = jnp.searchsorted(
        dst, jnp.arange(0, NPAD + 1, NPT, dtype=jnp.int32),
        side="left").astype(jnp.int32)
    elo = (bounds[:-1] // 768) * 768
    nmc = (bounds[1:] - elo + (MCH - 1)) // MCH

    x = jnp.concatenate([feat1, feat2], axis=1)
    xpad = jnp.pad(x, ((0, NPAD - N), (0, 0)))
    perm = jnp.asarray(_PERM)
    permbf = jnp.asarray(_PERM_BF)
    wall = jnp.concatenate([Wq[:, perm], Wk[:, permbf], Wv[:, permbf],
                            Wskip[:, perm]], axis=1)
    ball = jnp.concatenate([bq[perm], bk[permbf], bv[permbf],
                            bskip[perm]]).reshape(1, 4 * HIDDEN)
    q, kk, vv, skip = _project(xpad, wall, ball)

    kk32 = lax.bitcast_convert_type(kk.reshape(NPAD, 128, 2), jnp.int32)
    vv32 = lax.bitcast_convert_type(vv.reshape(NPAD, 128, 2), jnp.int32)
    msg = _sc_edge(q, kk32, vv32, src2, dw, elo, nmc)

    batch_pad = jnp.concatenate(
        [batch, jnp.full((NPAD - N,), NB, jnp.int32)]).reshape(NPAD, 1)
    pooled = _final(msg, skip, batch_pad, ln_w[perm].reshape(1, HIDDEN),
                    ln_b[perm].reshape(1, HIDDEN))
    return pooled[:, jnp.asarray(_IPERM)]


# speedup vs baseline: 1.1442x; 1.1442x over previous
import functools

import jax
import jax.numpy as jnp
from jax import lax
from jax.experimental import pallas as pl
from jax.experimental.pallas import tpu as pltpu, tpu_sc as plsc

N = 10000
NPAD = 10240
IN_CH = 128
HIDDEN = 256
HEADS = 8
DH = HIDDEN // HEADS
NEG_SLOPE = 0.01
EPS = 1e-5
NB = 64

E = 320000
NPT = 80
NRANGES = NPAD // NPT
C = 48
MCH = 16 * C
EPAD = 322560
INV_SQRT_DH = 1.0 / (DH ** 0.5)

NC = 2
NS = 16

import numpy as _np
_PERM = _np.array([(l % 8) * 32 + 2 * j + (l // 8)
                   for j in range(16) for l in range(16)], dtype=_np.int32)
_IPERM = _np.argsort(_PERM).astype(_np.int32)
_PERM_BF = _np.empty(256, dtype=_np.int32)
for _p in range(8):
    for _t in range(16):
        _PERM_BF[32 * _p + 2 * _t] = _PERM[16 * (2 * _p) + _t]
        _PERM_BF[32 * _p + 2 * _t + 1] = _PERM[16 * (2 * _p + 1) + _t]


def _sload(ref, i):
    return ref[pl.ds(i, 16)][0]


def _gather16(vec, idx):
    return lax.gather(
        vec, idx.reshape(16, 1),
        lax.GatherDimensionNumbers(offset_dims=(), collapsed_slice_dims=(0,),
                                   start_index_map=(0,)),
        (1,), mode=lax.GatherScatterMode.PROMISE_IN_BOUNDS)


def _take16(vec, h):
    return _gather16(vec, jnp.full((16,), h, jnp.int32))


def _unpack_bf16(xi):
    a = lax.bitcast_convert_type(lax.shift_left(xi, 16), jnp.float32)
    b = lax.bitcast_convert_type(jnp.bitwise_and(xi, jnp.int32(-65536)),
                                 jnp.float32)
    return a, b


def _allsum16(vec, perms):
    t = vec
    for p in perms:
        t = t + _gather16(t, p)
    return t


def _mm_body(x_ref, w_ref, b_ref, qo, ko, vo, so):
    r = jnp.dot(x_ref[...], w_ref[...], preferred_element_type=jnp.float32)
    r = r + b_ref[...]
    qo[...] = r[:, 0:256]
    ko[...] = r[:, 256:512].astype(jnp.bfloat16)
    vo[...] = r[:, 512:768].astype(jnp.bfloat16)
    so[...] = r[:, 768:1024]


def _project(x, wall, ball):
    blk = 512
    grid = (NPAD // blk,)
    out = pl.pallas_call(
        _mm_body,
        grid=grid,
        in_specs=[
            pl.BlockSpec((blk, IN_CH), lambda i: (i, 0)),
            pl.BlockSpec((IN_CH, 4 * HIDDEN), lambda i: (0, 0)),
            pl.BlockSpec((1, 4 * HIDDEN), lambda i: (0, 0)),
        ],
        out_specs=[pl.BlockSpec((blk, HIDDEN), lambda i: (i, 0))] * 4,
        out_shape=[jax.ShapeDtypeStruct((NPAD, HIDDEN), jnp.float32),
                   jax.ShapeDtypeStruct((NPAD, HIDDEN), jnp.bfloat16),
                   jax.ShapeDtypeStruct((NPAD, HIDDEN), jnp.bfloat16),
                   jax.ShapeDtypeStruct((NPAD, HIDDEN), jnp.float32)],
    )(x, wall, ball)
    return out


def _wid():
    return lax.axis_index("s") * NC + lax.axis_index("c")


def _sc_edge_body(q_hbm, k_hbm, v_hbm, src2_hbm, dw_hbm, elo_hbm,
                  nmc_hbm, out_hbm, q_loc, num, den, krows0, vrows0, krows1,
                  vrows1, srcv2, dwv, elov, nmcv, semk0, semv0, semk1, semv1):
    wid = _wid()
    pltpu.sync_copy(elo_hbm, elov.at[pl.ds(0, NRANGES)])
    pltpu.sync_copy(nmc_hbm, nmcv.at[pl.ds(0, NRANGES)])
    iota16 = lax.iota(jnp.int32, 16)
    zero16 = jnp.zeros((16,), jnp.float32)
    perm8 = iota16 ^ 8
    kbufs = (krows0, krows1)
    vbufs = (vrows0, vrows1)
    ksems = (semk0, semk1)
    vsems = (semv0, semv1)

    def range_body(rr, _):
        r = wid * 4 + rr
        nlo = r * NPT

        def zero_body(n, _z):
            for j in range(16):
                num[n, pl.ds(j * 16, 16)] = zero16
            den[n, :] = zero16
            return 0
        lax.fori_loop(0, NPT, zero_body, 0)

        pltpu.sync_copy(q_hbm.at[pl.ds(pl.multiple_of(nlo, 8), NPT)], q_loc)

        elo = _sload(elov, r)
        nmc = _sload(nmcv, r)

        def make_edge_body(c, kb, vb):
            def pair_body(ei, _z):
                dww = dwv[pl.ds(2 * (c * C) + 4 * ei, 16)]
                for half in range(2):
                    e = 2 * ei + half
                    dstl_raw = dww[2 * half] - nlo
                    w_e = dww[2 * half + 1].astype(jnp.float32)
                    inr = jnp.logical_and(dstl_raw >= 0, dstl_raw < NPT)
                    dstl = jnp.clip(dstl_raw, 0, NPT - 1)
                    wf = jnp.where(inr, w_e, 0.0)

                    prods = [None] * 16
                    for p in range(8):
                        ka, kbb = _unpack_bf16(
                            kb[e, pl.ds(p * 16, 16)])
                        prods[2 * p] = ka * q_loc[dstl,
                                                  pl.ds((2 * p) * 16, 16)]
                        prods[2 * p + 1] = kbb * q_loc[
                            dstl, pl.ds((2 * p + 1) * 16, 16)]
                    while len(prods) > 1:
                        prods = [prods[i] + prods[i + 1]
                                 for i in range(0, len(prods), 2)]
                    part = prods[0]
                    dup = part + _gather16(part, perm8)
                    evec = jnp.exp(dup * INV_SQRT_DH) * wf
                    den[dstl, :] = den[dstl, :] + evec
                    for p in range(8):
                        va, vbb = _unpack_bf16(
                            vb[e, pl.ds(p * 16, 16)])
                        j0 = 2 * p
                        num[dstl, pl.ds(j0 * 16, 16)] = (
                            num[dstl, pl.ds(j0 * 16, 16)] + evec * va)
                        num[dstl, pl.ds((j0 + 1) * 16, 16)] = (
                            num[dstl, pl.ds((j0 + 1) * 16, 16)] + evec * vbb)
                return 0
            return pair_body

        def meta_body(m, _z):
            moff = pl.multiple_of(elo + m * MCH, 384)
            mrow = pl.multiple_of(moff // 48, 8)
            pltpu.sync_copy(dw_hbm.at[pl.ds(pl.multiple_of(2 * moff, 8),
                                            2 * MCH)],
                            dwv.at[pl.ds(0, 2 * MCH)])
            pltpu.sync_copy(src2_hbm.at[pl.ds(mrow, 16)], srcv2)
            pend = [None, None]
            for c in range(16):
                b = c % 2
                if c == 0:
                    pend[0] = (
                        pltpu.async_copy(k_hbm.at[srcv2.at[0]], kbufs[0],
                                         ksems[0]),
                        pltpu.async_copy(v_hbm.at[srcv2.at[0]], vbufs[0],
                                         vsems[0]),
                    )
                if c < 15:
                    nb = (c + 1) % 2
                    pend[nb] = (
                        pltpu.async_copy(k_hbm.at[srcv2.at[c + 1]], kbufs[nb],
                                         ksems[nb]),
                        pltpu.async_copy(v_hbm.at[srcv2.at[c + 1]], vbufs[nb],
                                         vsems[nb]),
                    )
                pend[b][0].wait()
                pend[b][1].wait()
                lax.fori_loop(0, C // 2, make_edge_body(c, kbufs[b], vbufs[b]), 0)
            return 0

        lax.fori_loop(0, nmc, meta_body, 0)

        def div_body(n, _z):
            inv = 1.0 / (den[n, :] + 1e-16)
            for j in range(16):
                num[n, pl.ds(j * 16, 16)] = num[n, pl.ds(j * 16, 16)] * inv
            return 0
        lax.fori_loop(0, NPT, div_body, 0)

        pltpu.sync_copy(num, out_hbm.at[pl.ds(pl.multiple_of(nlo, 8), NPT)])
        return 0

    lax.fori_loop(0, 4, range_body, 0)


def _sc_edge(q, k, v, src2, dw, elo, nmc, interpret=False):
    mesh = plsc.VectorSubcoreMesh(core_axis_name="c", subcore_axis_name="s",
                                  num_cores=NC, num_subcores=NS)
    f = pl.kernel(
        _sc_edge_body,
        out_type=jax.ShapeDtypeStruct((NPAD, HIDDEN), jnp.float32),
        mesh=mesh,
        scratch_types=[
            pltpu.VMEM((NPT, HIDDEN), jnp.float32),
            pltpu.VMEM((NPT, HIDDEN), jnp.float32),
            pltpu.VMEM((NPT, 16), jnp.float32),
            pltpu.VMEM((C, 128), jnp.int32),
            pltpu.VMEM((C, 128), jnp.int32),
            pltpu.VMEM((C, 128), jnp.int32),
            pltpu.VMEM((C, 128), jnp.int32),
            pltpu.VMEM((16, C), jnp.int32),
            pltpu.VMEM((2 * MCH + 16,), jnp.int32),
            pltpu.VMEM((NRANGES + 16,), jnp.int32),
            pltpu.VMEM((NRANGES + 16,), jnp.int32),
            pltpu.SemaphoreType.DMA,
            pltpu.SemaphoreType.DMA,
            pltpu.SemaphoreType.DMA,
            pltpu.SemaphoreType.DMA,
        ],
        interpret=interpret,
    )
    return f(q, k, v, src2, dw, elo, nmc)


FBLK = 1024
NFB = NPAD // FBLK


def _final_body(batch_ref, msg_ref, skip_ref, lnw_ref, lnb_ref, out_ref,
                s1, s2, aux):
    p = pl.program_id(0)
    i = pl.program_id(1)

    @pl.when(jnp.logical_and(p == 0, i == 0))
    def _init():
        s1[...] = jnp.zeros_like(s1)
        s2[...] = jnp.zeros_like(s2)
        aux[...] = jnp.zeros_like(aux)

    out = msg_ref[...] + skip_ref[...]
    oh = (batch_ref[...] == lax.broadcasted_iota(jnp.int32, (FBLK, NB), 1))
    oh = oh.astype(jnp.float32)

    @pl.when(p == 0)
    def _pass0():
        s1[...] = s1[...] + lax.dot_general(
            oh, out, (((0,), (0,)), ((), ())),
            preferred_element_type=jnp.float32)
        s2[...] = s2[...] + lax.dot_general(
            oh, out * out, (((0,), (0,)), ((), ())),
            preferred_element_type=jnp.float32)
        aux[:, 0:1] = aux[:, 0:1] + jnp.sum(oh, axis=0)[:, None]

    @pl.when(jnp.logical_and(p == 0, i == NFB - 1))
    def _stats():
        cnt = aux[:, 0:1]
        normc = jnp.clip(cnt, 1.0, None) * float(HIDDEN)
        mean = jnp.sum(s1[...], axis=1)[:, None] / normc
        ms = jnp.sum(s2[...], axis=1)[:, None] / normc
        var = ms - mean * mean
        inv = lax.rsqrt(var + EPS)
        aux[:, 1:2] = mean
        aux[:, 2:3] = inv
        s2[...] = jnp.zeros_like(s2)

    @pl.when(p == 1)
    def _pass1():
        mean_row = jnp.dot(oh, aux[:, 1:2],
                           preferred_element_type=jnp.float32)
        inv_row = jnp.dot(oh, aux[:, 2:3],
                          preferred_element_type=jnp.float32)
        y = (out - mean_row) * inv_row
        y = y * lnw_ref[...] + lnb_ref[...]
        y = jnp.where(y >= 0, y, NEG_SLOPE * y)
        s2[...] = s2[...] + lax.dot_general(
            oh, y, (((0,), (0,)), ((), ())),
            preferred_element_type=jnp.float32)

    @pl.when(jnp.logical_and(p == 1, i == NFB - 1))
    def _emit():
        out_ref[...] = s2[...] / jnp.clip(aux[:, 0:1], 1.0, None)


def _final(msg, skip, batch_pad, lnw, lnb):
    return pl.pallas_call(
        _final_body,
        grid=(2, NFB),
        in_specs=[
            pl.BlockSpec((FBLK, 1), lambda p, i: (i, 0)),
            pl.BlockSpec((FBLK, HIDDEN), lambda p, i: (i, 0)),
            pl.BlockSpec((FBLK, HIDDEN), lambda p, i: (i, 0)),
            pl.BlockSpec((1, HIDDEN), lambda p, i: (0, 0)),
            pl.BlockSpec((1, HIDDEN), lambda p, i: (0, 0)),
        ],
        out_specs=pl.BlockSpec((NB, HIDDEN), lambda p, i: (0, 0)),
        out_shape=jax.ShapeDtypeStruct((NB, HIDDEN), jnp.float32),
        scratch_shapes=[
            pltpu.VMEM((NB, HIDDEN), jnp.float32),
            pltpu.VMEM((NB, HIDDEN), jnp.float32),
            pltpu.VMEM((NB, 128), jnp.float32),
        ],
    )(batch_pad, msg, skip, lnw, lnb)


def kernel(feat1, feat2, edge_index1, edge_index2, batch, Wq, bq, Wk, bk,
           Wv, bv, Wskip, bskip, ln_w, ln_b):
    ei = jnp.concatenate([edge_index1, edge_index2], axis=1)
    code = ei[1] * jnp.int32(N) + ei[0]
    scode = jnp.sort(code)
    keep = jnp.concatenate(
        [jnp.ones((1,), bool), scode[1:] != scode[:-1]])
    src = scode % jnp.int32(N)
    dst = scode // jnp.int32(N)
    w = keep.astype(jnp.float32)
    padn = EPAD - E
    src_p = jnp.concatenate([src, jnp.zeros((padn,), jnp.int32)])
    dst_p = jnp.concatenate([dst, jnp.full((padn,), N, jnp.int32)])
    w_i = jnp.concatenate([keep.astype(jnp.int32),
                           jnp.zeros((padn,), jnp.int32)])
    src2 = src_p.reshape(EPAD // C, C)
    dw = jnp.stack([dst_p, w_i], axis=1).reshape(2 * EPAD)
    bounds = jnp.searchsorted(
        dst, jnp.arange(0, NPAD + 1, NPT, dtype=jnp.int32),
        side="left").astype(jnp.int32)
    elo = (bounds[:-1] // 384) * 384
    nmc = (bounds[1:] - elo + (MCH - 1)) // MCH

    x = jnp.concatenate([feat1, feat2], axis=1)
    xpad = jnp.pad(x, ((0, NPAD - N), (0, 0)))
    perm = jnp.asarray(_PERM)
    permbf = jnp.asarray(_PERM_BF)
    wall = jnp.concatenate([Wq[:, perm], Wk[:, permbf], Wv[:, permbf],
                            Wskip[:, perm]], axis=1)
    ball = jnp.concatenate([bq[perm], bk[permbf], bv[permbf],
                            bskip[perm]]).reshape(1, 4 * HIDDEN)
    q, kk, vv, skip = _project(xpad, wall, ball)

    kk32 = lax.bitcast_convert_type(kk.reshape(NPAD, 128, 2), jnp.int32)
    vv32 = lax.bitcast_convert_type(vv.reshape(NPAD, 128, 2), jnp.int32)
    msg = _sc_edge(q, kk32, vv32, src2, dw, elo, nmc)

    batch_pad = jnp.concatenate(
        [batch, jnp.full((NPAD - N,), NB, jnp.int32)]).reshape(NPAD, 1)
    pooled = _final(msg, skip, batch_pad, ln_w[perm].reshape(1, HIDDEN),
                    ln_b[perm].reshape(1, HIDDEN))
    return pooled[:, jnp.asarray(_IPERM)]
